# baseline (speedup 1.0000x reference)
import functools
import math

import jax
import jax.numpy as jnp
from jax import lax
from jax.experimental import pallas as pl
from jax.experimental.pallas import tpu as pltpu
from jax.experimental.pallas import tpu_sc as plsc

B = 1024
D = 64
N = 100000
K = 56
GSZ = 64
CT = 2048
NP = 102400
NT = NP // CT
GPT = CT // GSZ
NG = NP // GSZ
NC = K * GSZ
PR = 128
NPR = NP // PR
NEG = -3.0e38

NW = 32
RPW = B // NW
SPW = RPW * K
CHUNK = 128
NCH = SPW // CHUNK
NRT = RPW // 8
CW = 12800
NCC = 8
NALN = (N // PR) * PR
CWL = NALN - (NCC - 1) * CW
NVR = SPW // 16


def _k1_body(img_ref, pbt_ref, wq_ref, wk_ref, lg_ref, gm_ref, q_scr):
    j = pl.program_id(0)

    @pl.when(j == 0)
    def _():
        q_scr[...] = lax.dot_general(
            img_ref[...], wq_ref[...], (((1,), (1,)), ((), ())),
            preferred_element_type=jnp.float32) * (1.0 / math.sqrt(D))

    kt = lax.dot_general(wk_ref[...], pbt_ref[...], (((1,), (0,)), ((), ())),
                         preferred_element_type=jnp.float32)
    lg = lax.dot_general(q_scr[...], kt, (((1,), (0,)), ((), ())),
                         preferred_element_type=jnp.float32)
    col = j * CT + lax.broadcasted_iota(jnp.int32, (B, CT), 1)
    lg = jnp.where(col < N, lg, NEG)
    for t in range(CT // PR):
        lg_ref[:, t, :] = lg[:, t * PR:(t + 1) * PR]
    for g in range(GPT):
        gm_ref[0, :, g] = jnp.max(lg[:, g * GSZ:(g + 1) * GSZ], axis=1)


_k1 = pl.pallas_call(
    _k1_body,
    grid=(NT,),
    in_specs=[
        pl.BlockSpec((B, D), lambda j: (0, 0)),
        pl.BlockSpec((D, CT), lambda j: (0, j)),
        pl.BlockSpec((D, D), lambda j: (0, 0)),
        pl.BlockSpec((D, D), lambda j: (0, 0)),
    ],
    out_specs=[
        pl.BlockSpec((B, CT // PR, PR), lambda j: (0, j, 0)),
        pl.BlockSpec((1, B, GPT), lambda j: (j, 0, 0)),
    ],
    out_shape=[
        jax.ShapeDtypeStruct((B, NP // PR, PR), jnp.float32),
        jax.ShapeDtypeStruct((NT, B, GPT), jnp.float32),
    ],
    scratch_shapes=[pltpu.VMEM((B, D), jnp.float32)],
)


def _k2_body(gm_ref, gsel_ref, gm_scr):
    it = pl.program_id(0)

    @pl.when(it == 0)
    def _():
        gm_scr[...] = gm_ref[...]

    gm = gm_scr[...]
    giota = lax.broadcasted_iota(jnp.int32, (B, NG), 1).astype(jnp.float32)
    m = jnp.max(gm, axis=1)
    gi = jnp.min(jnp.where(gm == m[:, None], giota, 3.0e38), axis=1)
    gsel_ref[0] = gi.astype(jnp.int32)[:, None]
    gm_scr[...] = jnp.where(giota == gi[:, None], NEG, gm)


_k2 = pl.pallas_call(
    _k2_body,
    grid=(K,),
    in_specs=[pl.BlockSpec((B, NG), lambda it: (0, 0))],
    out_specs=pl.BlockSpec((1, B, 1), lambda it: (it, 0, 0)),
    out_shape=jax.ShapeDtypeStruct((K, B, 1), jnp.int32),
    scratch_shapes=[pltpu.VMEM((B, NG), jnp.float32)],
)


def _k3_body(table_ref, gidx_ref, out_ref, idx_v, buf_v, sem):
    wid = lax.axis_index("s") * 2 + lax.axis_index("c")
    pltpu.sync_copy(gidx_ref.at[wid], idx_v)
    for c in range(NCH):
        pltpu.async_copy(table_ref.at[idx_v.at[c]], buf_v, sem).wait()
        pltpu.sync_copy(buf_v, out_ref.at[pl.ds(wid * SPW + c * CHUNK, CHUNK)])


@functools.cache
def _k3():
    return pl.kernel(
        _k3_body,
        out_type=jax.ShapeDtypeStruct((B * K, PR), jnp.float32),
        mesh=plsc.VectorSubcoreMesh(core_axis_name="c", subcore_axis_name="s"),
        scratch_types=[
            pltpu.VMEM((NCH, CHUNK), jnp.int32),
            pltpu.VMEM((CHUNK, PR), jnp.float32),
            pltpu.SemaphoreType.DMA,
        ],
    )


RB = 256


def _k4_body(cand_ref, gsel_ref, ti_ref, sm_ref, cand_scr, tv_scr, ti_scr):
    it = pl.program_id(1)

    @pl.when(it == 0)
    def _():
        cand3 = cand_ref[...]
        gsel0 = gsel_ref[...]
        par = (gsel0 % 2)[:, :, None]
        cand_scr[...] = jnp.where(par == 1, cand3[:, :, GSZ:], cand3[:, :, :GSZ])

    BIGO = 1 << 20
    cand = cand_scr[...]
    m2 = jnp.max(cand, axis=2)
    m = jnp.max(m2, axis=1)
    eq = cand == m[:, None, None]
    o_iota = lax.broadcasted_iota(jnp.int32, (RB, K, GSZ), 2)
    so = jnp.min(jnp.where(eq, o_iota, BIGO), axis=2)
    gsel = gsel_ref[...]
    flatc = gsel * GSZ + so
    fi = jnp.min(jnp.where(so < BIGO, flatc, 1 << 30), axis=1)
    mask = ((flatc[:, :, None] == fi[:, None, None])
            & (o_iota == so[:, :, None]))
    cand_scr[...] = jnp.where(mask, NEG, cand)
    sub = lax.broadcasted_iota(jnp.int32, (K, RB), 0)
    tv_scr[...] = jnp.where(sub == it, m[None, :], tv_scr[...])
    ti_scr[...] = jnp.where(sub == it, fi[None, :], ti_scr[...])

    @pl.when(it == K - 1)
    def _():
        tv = tv_scr[...]
        e = jnp.exp(tv - tv[0:1, :])
        sm_ref[...] = e / jnp.sum(e, axis=0)[None, :]
        ti_ref[...] = ti_scr[...]


_k4 = pl.pallas_call(
    _k4_body,
    grid=(B // RB, K),
    in_specs=[
        pl.BlockSpec((RB, K, PR), lambda b, it: (b, 0, 0)),
        pl.BlockSpec((RB, K), lambda b, it: (b, 0)),
    ],
    out_specs=[
        pl.BlockSpec((K, RB), lambda b, it: (0, b)),
        pl.BlockSpec((K, RB), lambda b, it: (0, b)),
    ],
    out_shape=[
        jax.ShapeDtypeStruct((K, B), jnp.int32),
        jax.ShapeDtypeStruct((K, B), jnp.float32),
    ],
    scratch_shapes=[
        pltpu.VMEM((RB, K, GSZ), jnp.float32),
        pltpu.VMEM((K, RB), jnp.float32),
        pltpu.VMEM((K, RB), jnp.int32),
    ],
)


def _k5_body(ci_ref, ri_ref, sm_ref, zrow_ref, attn_ref, c_v, r_v, v_v, zbuf):
    wid = lax.axis_index("s") * 2 + lax.axis_index("c")
    pltpu.sync_copy(ci_ref.at[wid], c_v)
    pltpu.sync_copy(ri_ref.at[wid], r_v)
    pltpu.sync_copy(sm_ref.at[wid], v_v)
    pltpu.sync_copy(zrow_ref, zbuf)
    zero16 = jnp.zeros((16,), jnp.float32)
    for rt in range(NRT):
        grt = wid * NRT + rt
        for cc in range(NCC):
            c0 = cc * CW
            cw = CW if cc < NCC - 1 else CWL

            def mk(use_vals, c0=c0, cw=cw, grt=grt):
                def body(i, carry):
                    c16 = c_v[pl.ds(i * 16, 16)]
                    r16 = r_v[pl.ds(i * 16, 16)]
                    m = ((lax.shift_right_logical(r16, 3) == grt)
                         & (c16 >= c0) & (c16 < c0 + cw))
                    ir = r16 & 7
                    ic = jnp.minimum(jnp.maximum(c16 - c0, 0), cw - 1)
                    x = v_v[pl.ds(i * 16, 16)] if use_vals else zero16
                    plsc.store_scatter(zbuf, [ir, ic], x, mask=m)
                    return carry
                return body

            lax.fori_loop(0, NVR, mk(True), 0)
            pltpu.sync_copy(zbuf.at[:, pl.ds(0, cw)],
                            attn_ref.at[pl.ds(grt * 8, 8), pl.ds(c0, cw)])
            lax.fori_loop(0, NVR, mk(False), 0)


@functools.cache
def _k5():
    return pl.kernel(
        _k5_body,
        out_type=jax.ShapeDtypeStruct((B, N), jnp.float32),
        mesh=plsc.VectorSubcoreMesh(core_axis_name="c", subcore_axis_name="s"),
        compiler_params=pltpu.CompilerParams(needs_layout_passes=False),
        scratch_types=[
            pltpu.VMEM((SPW,), jnp.int32),
            pltpu.VMEM((SPW,), jnp.int32),
            pltpu.VMEM((SPW,), jnp.float32),
            pltpu.VMEM((8, CW), jnp.float32),
        ],
    )


def _k6_body(ti_ref, sm_ref, attn_ref, out_ref):
    del attn_ref
    ti = ti_ref[...]
    sm = sm_ref[...]
    col = NALN + lax.broadcasted_iota(jnp.int32, (B, PR), 1)
    acc = jnp.zeros((B, PR), jnp.float32)
    for k in range(K):
        acc = acc + jnp.where(col == ti[:, k][:, None], sm[:, k][:, None], 0.0)
    out_ref[...] = acc


_k6 = pl.pallas_call(
    _k6_body,
    grid=(1,),
    in_specs=[
        pl.BlockSpec((B, K), lambda i: (0, 0)),
        pl.BlockSpec((B, K), lambda i: (0, 0)),
        pl.BlockSpec((8, PR), lambda i: (0, 0)),
    ],
    out_specs=pl.BlockSpec((B, PR), lambda i: (0, NALN // PR)),
    out_shape=jax.ShapeDtypeStruct((B, N), jnp.float32),
    input_output_aliases={2: 0},
)


def kernel(img_emb, prompt_bank, Wq, Wk):
    pbt = jnp.pad(prompt_bank, ((0, NP - N), (0, 0))).T
    logits3, gm3 = _k1(img_emb, pbt, Wq, Wk)
    gmax = gm3.transpose(1, 0, 2).reshape(B, NG)
    gsel = _k2(gmax)[:, :, 0].T
    gidx = gsel // 2 + jnp.arange(B, dtype=jnp.int32)[:, None] * NPR
    cand = _k3()(logits3.reshape(B * NPR, PR), gidx.reshape(NW, NCH, CHUNK))
    ti3, sm3 = _k4(cand.reshape(B, K, PR), gsel)
    top_idxs = ti3.T
    sm = sm3.T
    rows = jnp.broadcast_to(jnp.arange(B, dtype=jnp.int32)[:, None], (B, K))
    zrow = jnp.zeros((8, CW), jnp.float32)
    attn = _k5()(top_idxs.reshape(NW, SPW), rows.reshape(NW, SPW),
                 sm.reshape(NW, SPW), zrow)
    attn = _k6(top_idxs, sm, attn)
    return attn, top_idxs


# speedup vs baseline: 1.6297x; 1.6297x over previous
import functools
import math

import jax
import jax.numpy as jnp
from jax import lax
from jax.experimental import pallas as pl
from jax.experimental.pallas import tpu as pltpu
from jax.experimental.pallas import tpu_sc as plsc

B = 1024
D = 64
N = 100000
K = 56
GSZ = 64
CT = 2048
NP = 102400
NT = NP // CT
GPT = CT // GSZ
NG = NP // GSZ
NC = K * GSZ
PR = 128
NPR = NP // PR
NEG = -3.0e38

NW = 32
RPW = B // NW
SPW = RPW * K
CHUNK = 128
NCH = SPW // CHUNK
NRT = RPW // 8
CW = 12800
NCC = 8
NALN = (N // PR) * PR
CWL = NALN - (NCC - 1) * CW
NVR = SPW // 16


def _k1_body(img_ref, pbt_ref, wq_ref, wk_ref, lg_ref, gm_ref, q_scr):
    j = pl.program_id(0)

    @pl.when(j == 0)
    def _():
        q_scr[...] = lax.dot_general(
            img_ref[...], wq_ref[...], (((1,), (1,)), ((), ())),
            preferred_element_type=jnp.float32) * (1.0 / math.sqrt(D))

    kt = lax.dot_general(wk_ref[...], pbt_ref[...], (((1,), (0,)), ((), ())),
                         preferred_element_type=jnp.float32)
    lg = lax.dot_general(q_scr[...], kt, (((1,), (0,)), ((), ())),
                         preferred_element_type=jnp.float32)
    col = j * CT + lax.broadcasted_iota(jnp.int32, (B, CT), 1)
    lg = jnp.where(col < N, lg, NEG)
    for t in range(CT // PR):
        lg_ref[:, t, :] = lg[:, t * PR:(t + 1) * PR]
    for g in range(GPT):
        gm_ref[0, :, g] = jnp.max(lg[:, g * GSZ:(g + 1) * GSZ], axis=1)


_k1 = pl.pallas_call(
    _k1_body,
    grid=(NT,),
    in_specs=[
        pl.BlockSpec((B, D), lambda j: (0, 0)),
        pl.BlockSpec((D, CT), lambda j: (0, j)),
        pl.BlockSpec((D, D), lambda j: (0, 0)),
        pl.BlockSpec((D, D), lambda j: (0, 0)),
    ],
    out_specs=[
        pl.BlockSpec((B, CT // PR, PR), lambda j: (0, j, 0)),
        pl.BlockSpec((1, B, GPT), lambda j: (j, 0, 0)),
    ],
    out_shape=[
        jax.ShapeDtypeStruct((B, NP // PR, PR), jnp.float32),
        jax.ShapeDtypeStruct((NT, B, GPT), jnp.float32),
    ],
    scratch_shapes=[pltpu.VMEM((B, D), jnp.float32)],
)


def _k2_body(gm_ref, gsel_ref, gm_scr):
    it = pl.program_id(0)

    @pl.when(it == 0)
    def _():
        gm_scr[...] = gm_ref[...]

    gm = gm_scr[...]
    giota = lax.broadcasted_iota(jnp.int32, (B, NG), 1).astype(jnp.float32)
    m = jnp.max(gm, axis=1)
    gi = jnp.min(jnp.where(gm == m[:, None], giota, 3.0e38), axis=1)
    gsel_ref[0] = gi.astype(jnp.int32)[:, None]
    gm_scr[...] = jnp.where(giota == gi[:, None], NEG, gm)


_k2 = pl.pallas_call(
    _k2_body,
    grid=(K,),
    in_specs=[pl.BlockSpec((B, NG), lambda it: (0, 0))],
    out_specs=pl.BlockSpec((1, B, 1), lambda it: (it, 0, 0)),
    out_shape=jax.ShapeDtypeStruct((K, B, 1), jnp.int32),
    scratch_shapes=[pltpu.VMEM((B, NG), jnp.float32)],
)


def _k3_body(table_ref, gidx_ref, out_ref, idx_v, buf_v, sem):
    wid = lax.axis_index("s") * 2 + lax.axis_index("c")
    pltpu.sync_copy(gidx_ref.at[wid], idx_v)
    for c in range(NCH):
        pltpu.async_copy(table_ref.at[idx_v.at[c]], buf_v, sem).wait()
        pltpu.sync_copy(buf_v, out_ref.at[pl.ds(wid * SPW + c * CHUNK, CHUNK)])


@functools.cache
def _k3():
    return pl.kernel(
        _k3_body,
        out_type=jax.ShapeDtypeStruct((B * K, PR), jnp.float32),
        mesh=plsc.VectorSubcoreMesh(core_axis_name="c", subcore_axis_name="s"),
        scratch_types=[
            pltpu.VMEM((NCH, CHUNK), jnp.int32),
            pltpu.VMEM((CHUNK, PR), jnp.float32),
            pltpu.SemaphoreType.DMA,
        ],
    )


RB = 256


def _k4_body(cand_ref, gsel_ref, ti_ref, sm_ref, cand_scr, cidx_scr, tv_scr, ti_scr):
    it = pl.program_id(1)

    @pl.when(it == 0)
    def _():
        cand3 = cand_ref[...]
        gsel = gsel_ref[...]
        par = (gsel % 2)[:, :, None]
        cand_scr[...] = jnp.where(par == 1, cand3[:, :, GSZ:], cand3[:, :, :GSZ])
        o_iota = lax.broadcasted_iota(jnp.int32, (RB, K, GSZ), 2)
        cidx_scr[...] = (gsel[:, :, None] * GSZ + o_iota).astype(jnp.float32)

    cidx = cidx_scr[...]
    cand = cand_scr[...]
    m = jnp.max(jnp.max(cand, axis=2), axis=1)
    fi3 = jnp.where(cand == m[:, None, None], cidx, 3.0e38)
    fi = jnp.min(jnp.min(fi3, axis=2), axis=1)
    cand_scr[...] = jnp.where(cidx == fi[:, None, None], NEG, cand)
    sub = lax.broadcasted_iota(jnp.int32, (K, RB), 0)
    tv_scr[...] = jnp.where(sub == it, m[None, :], tv_scr[...])
    ti_scr[...] = jnp.where(sub == it, fi[None, :], ti_scr[...])

    @pl.when(it == K - 1)
    def _():
        tv = tv_scr[...]
        e = jnp.exp(tv - tv[0:1, :])
        sm_ref[...] = e / jnp.sum(e, axis=0)[None, :]
        ti_ref[...] = ti_scr[...].astype(jnp.int32)


_k4 = pl.pallas_call(
    _k4_body,
    grid=(B // RB, K),
    in_specs=[
        pl.BlockSpec((RB, K, PR), lambda b, it: (b, 0, 0)),
        pl.BlockSpec((RB, K), lambda b, it: (b, 0)),
    ],
    out_specs=[
        pl.BlockSpec((K, RB), lambda b, it: (0, b)),
        pl.BlockSpec((K, RB), lambda b, it: (0, b)),
    ],
    out_shape=[
        jax.ShapeDtypeStruct((K, B), jnp.int32),
        jax.ShapeDtypeStruct((K, B), jnp.float32),
    ],
    scratch_shapes=[
        pltpu.VMEM((RB, K, GSZ), jnp.float32),
        pltpu.VMEM((RB, K, GSZ), jnp.float32),
        pltpu.VMEM((K, RB), jnp.float32),
        pltpu.VMEM((K, RB), jnp.float32),
    ],
)


def _k5_body(ci_ref, ri_ref, sm_ref, zrow_ref, attn_ref, c_v, r_v, v_v, zbuf):
    wid = lax.axis_index("s") * 2 + lax.axis_index("c")
    pltpu.sync_copy(ci_ref.at[wid], c_v)
    pltpu.sync_copy(ri_ref.at[wid], r_v)
    pltpu.sync_copy(sm_ref.at[wid], v_v)
    pltpu.sync_copy(zrow_ref, zbuf)
    zero16 = jnp.zeros((16,), jnp.float32)
    for rt in range(NRT):
        grt = wid * NRT + rt
        for cc in range(NCC):
            c0 = cc * CW
            cw = CW if cc < NCC - 1 else CWL

            def mk(use_vals, c0=c0, cw=cw, grt=grt):
                def body(i, carry):
                    c16 = c_v[pl.ds(i * 16, 16)]
                    r16 = r_v[pl.ds(i * 16, 16)]
                    m = ((lax.shift_right_logical(r16, 3) == grt)
                         & (c16 >= c0) & (c16 < c0 + cw))
                    ir = r16 & 7
                    ic = jnp.minimum(jnp.maximum(c16 - c0, 0), cw - 1)
                    x = v_v[pl.ds(i * 16, 16)] if use_vals else zero16
                    plsc.store_scatter(zbuf, [ir, ic], x, mask=m)
                    return carry
                return body

            lax.fori_loop(0, NVR, mk(True), 0)
            pltpu.sync_copy(zbuf.at[:, pl.ds(0, cw)],
                            attn_ref.at[pl.ds(grt * 8, 8), pl.ds(c0, cw)])
            lax.fori_loop(0, NVR, mk(False), 0)


@functools.cache
def _k5():
    return pl.kernel(
        _k5_body,
        out_type=jax.ShapeDtypeStruct((B, N), jnp.float32),
        mesh=plsc.VectorSubcoreMesh(core_axis_name="c", subcore_axis_name="s"),
        compiler_params=pltpu.CompilerParams(needs_layout_passes=False),
        scratch_types=[
            pltpu.VMEM((SPW,), jnp.int32),
            pltpu.VMEM((SPW,), jnp.int32),
            pltpu.VMEM((SPW,), jnp.float32),
            pltpu.VMEM((8, CW), jnp.float32),
        ],
    )


def _k6_body(ti_ref, sm_ref, attn_ref, out_ref):
    del attn_ref
    ti = ti_ref[...]
    sm = sm_ref[...]
    col = NALN + lax.broadcasted_iota(jnp.int32, (B, PR), 1)
    acc = jnp.zeros((B, PR), jnp.float32)
    for k in range(K):
        acc = acc + jnp.where(col == ti[:, k][:, None], sm[:, k][:, None], 0.0)
    out_ref[...] = acc


_k6 = pl.pallas_call(
    _k6_body,
    grid=(1,),
    in_specs=[
        pl.BlockSpec((B, K), lambda i: (0, 0)),
        pl.BlockSpec((B, K), lambda i: (0, 0)),
        pl.BlockSpec((8, PR), lambda i: (0, 0)),
    ],
    out_specs=pl.BlockSpec((B, PR), lambda i: (0, NALN // PR)),
    out_shape=jax.ShapeDtypeStruct((B, N), jnp.float32),
    input_output_aliases={2: 0},
)


def kernel(img_emb, prompt_bank, Wq, Wk):
    pbt = jnp.pad(prompt_bank, ((0, NP - N), (0, 0))).T
    logits3, gm3 = _k1(img_emb, pbt, Wq, Wk)
    gmax = gm3.transpose(1, 0, 2).reshape(B, NG)
    gsel = _k2(gmax)[:, :, 0].T
    gidx = gsel // 2 + jnp.arange(B, dtype=jnp.int32)[:, None] * NPR
    cand = _k3()(logits3.reshape(B * NPR, PR), gidx.reshape(NW, NCH, CHUNK))
    ti3, sm3 = _k4(cand.reshape(B, K, PR), gsel)
    top_idxs = ti3.T
    sm = sm3.T
    rows = jnp.broadcast_to(jnp.arange(B, dtype=jnp.int32)[:, None], (B, K))
    zrow = jnp.zeros((8, CW), jnp.float32)
    attn = _k5()(top_idxs.reshape(NW, SPW), rows.reshape(NW, SPW),
                 sm.reshape(NW, SPW), zrow)
    attn = _k6(top_idxs, sm, attn)
    return attn, top_idxs
